# baseline (speedup 1.0000x reference)
import functools

import jax
import jax.numpy as jnp
from jax.experimental import pallas as pl
from jax.experimental.pallas import tpu as pltpu

_TEMP = 0.05
_BATCH = 1024
_FEAT = 128
_N = 100000
_TILE = 2048


def _fused_kernel(x_ref, tgt_ref, f_ref, out_ref, loss_ref,
                  m_ref, s_ref, t_ref, *, n_tiles):
    j = pl.program_id(0)

    @pl.when(j == 0)
    def _init():
        m_ref[...] = jnp.full_like(m_ref, -jnp.inf)
        s_ref[...] = jnp.zeros_like(s_ref)
        t_ref[...] = jnp.zeros_like(t_ref)

    x = x_ref[...]
    norm = jnp.sqrt(jnp.sum(x * x, axis=1, keepdims=True))
    xn = x / jnp.maximum(norm, 1e-12)

    logits = jax.lax.dot_general(
        xn, f_ref[...],
        dimension_numbers=(((1,), (1,)), ((), ())),
        preferred_element_type=jnp.float32,
    ) * (1.0 / _TEMP)
    out_ref[...] = logits

    col = j * _TILE + jax.lax.broadcasted_iota(jnp.int32, (_BATCH, _TILE), 1)
    valid = col < _N
    masked = jnp.where(valid, logits, -jnp.inf)

    m_old = m_ref[...]
    m_j = jnp.max(masked, axis=1, keepdims=True)
    m_new = jnp.maximum(m_old, m_j)
    s_ref[...] = (s_ref[...] * jnp.exp(m_old - m_new)
                  + jnp.sum(jnp.exp(masked - m_new), axis=1, keepdims=True))
    m_ref[...] = m_new

    is_tgt = col == tgt_ref[...]
    t_ref[...] += jnp.sum(jnp.where(is_tgt, logits, 0.0), axis=1,
                          keepdims=True)

    @pl.when(j == n_tiles - 1)
    def _finish():
        logz = m_ref[...] + jnp.log(s_ref[...])
        loss_ref[...] = jnp.mean(logz - t_ref[...]).reshape(1, 1)


def kernel(inputs, targets, features):
    n_tiles = pl.cdiv(_N, _TILE)
    tgt2d = targets.astype(jnp.int32).reshape(_BATCH, 1)

    outputs, loss = pl.pallas_call(
        functools.partial(_fused_kernel, n_tiles=n_tiles),
        grid=(n_tiles,),
        in_specs=[
            pl.BlockSpec((_BATCH, _FEAT), lambda j: (0, 0)),
            pl.BlockSpec((_BATCH, 1), lambda j: (0, 0)),
            pl.BlockSpec((_TILE, _FEAT), lambda j: (j, 0)),
        ],
        out_specs=[
            pl.BlockSpec((_BATCH, _TILE), lambda j: (0, j)),
            pl.BlockSpec((1, 1), lambda j: (0, 0)),
        ],
        out_shape=[
            jax.ShapeDtypeStruct((_BATCH, _N), jnp.float32),
            jax.ShapeDtypeStruct((1, 1), jnp.float32),
        ],
        scratch_shapes=[
            pltpu.VMEM((_BATCH, 1), jnp.float32),
            pltpu.VMEM((_BATCH, 1), jnp.float32),
            pltpu.VMEM((_BATCH, 1), jnp.float32),
        ],
    )(inputs, tgt2d, features)

    return (loss.reshape(()), outputs)


# speedup vs baseline: 1.0969x; 1.0969x over previous
import functools

import jax
import jax.numpy as jnp
from jax.experimental import pallas as pl
from jax.experimental.pallas import tpu as pltpu

_TEMP = 0.05
_BATCH = 1024
_FEAT = 128
_N = 100000
_TILE = 2048


def _fused_kernel(x_ref, f_ref, tf_ref, out_ref, loss_ref,
                  xn_ref, s_ref, *, n_tiles):
    j = pl.program_id(0)

    @pl.when(j == 0)
    def _init():
        x = x_ref[...]
        norm = jnp.sqrt(jnp.sum(x * x, axis=1, keepdims=True))
        xn_ref[...] = x / jnp.maximum(norm, 1e-12)
        s_ref[...] = jnp.zeros_like(s_ref)

    xn = xn_ref[...]
    logits = jax.lax.dot_general(
        xn, f_ref[...],
        dimension_numbers=(((1,), (1,)), ((), ())),
        preferred_element_type=jnp.float32,
    ) * (1.0 / _TEMP)
    out_ref[...] = logits
    e = jnp.exp(logits)

    @pl.when(j < n_tiles - 1)
    def _accum():
        s_ref[...] += jnp.sum(e, axis=1, keepdims=True)

    @pl.when(j == n_tiles - 1)
    def _finish():
        col = j * _TILE + jax.lax.broadcasted_iota(
            jnp.int32, (_BATCH, _TILE), 1)
        e_last = jnp.where(col < _N, e, 0.0)
        s = s_ref[...] + jnp.sum(e_last, axis=1, keepdims=True)
        tgt_logit = jnp.sum(xn * tf_ref[...], axis=1,
                            keepdims=True) * (1.0 / _TEMP)
        loss_ref[...] = jnp.mean(jnp.log(s) - tgt_logit).reshape(1, 1)


def kernel(inputs, targets, features):
    n_tiles = pl.cdiv(_N, _TILE)
    tgt_rows = jnp.take(features, targets.astype(jnp.int32), axis=0)

    outputs, loss = pl.pallas_call(
        functools.partial(_fused_kernel, n_tiles=n_tiles),
        grid=(n_tiles,),
        in_specs=[
            pl.BlockSpec((_BATCH, _FEAT), lambda j: (0, 0)),
            pl.BlockSpec((_TILE, _FEAT), lambda j: (j, 0)),
            pl.BlockSpec((_BATCH, _FEAT), lambda j: (0, 0)),
        ],
        out_specs=[
            pl.BlockSpec((_BATCH, _TILE), lambda j: (0, j)),
            pl.BlockSpec((1, 1), lambda j: (0, 0)),
        ],
        out_shape=[
            jax.ShapeDtypeStruct((_BATCH, _N), jnp.float32),
            jax.ShapeDtypeStruct((1, 1), jnp.float32),
        ],
        scratch_shapes=[
            pltpu.VMEM((_BATCH, _FEAT), jnp.float32),
            pltpu.VMEM((_BATCH, 1), jnp.float32),
        ],
    )(inputs, features, tgt_rows)

    return (loss.reshape(()), outputs)


# speedup vs baseline: 1.1046x; 1.0071x over previous
import functools

import jax
import jax.numpy as jnp
from jax.experimental import pallas as pl
from jax.experimental.pallas import tpu as pltpu

_TEMP = 0.05
_BATCH = 1024
_FEAT = 128
_N = 100000
_TILE = 4096


def _fused_kernel(x_ref, f_ref, tf_ref, out_ref, loss_ref,
                  xn_ref, s_ref, *, n_tiles):
    j = pl.program_id(0)

    @pl.when(j == 0)
    def _init():
        x = x_ref[...]
        norm = jnp.sqrt(jnp.sum(x * x, axis=1, keepdims=True))
        xn_ref[...] = x / jnp.maximum(norm, 1e-12)
        s_ref[...] = jnp.zeros_like(s_ref)

    xn = xn_ref[...]
    logits = jax.lax.dot_general(
        xn, f_ref[...],
        dimension_numbers=(((1,), (1,)), ((), ())),
        preferred_element_type=jnp.float32,
    ) * (1.0 / _TEMP)
    out_ref[...] = logits
    e = jnp.exp(logits)

    @pl.when(j < n_tiles - 1)
    def _accum():
        s_ref[...] += jnp.sum(e, axis=1, keepdims=True)

    @pl.when(j == n_tiles - 1)
    def _finish():
        col = j * _TILE + jax.lax.broadcasted_iota(
            jnp.int32, (_BATCH, _TILE), 1)
        e_last = jnp.where(col < _N, e, 0.0)
        s = s_ref[...] + jnp.sum(e_last, axis=1, keepdims=True)
        tgt_logit = jnp.sum(xn * tf_ref[...], axis=1,
                            keepdims=True) * (1.0 / _TEMP)
        loss_ref[...] = jnp.mean(jnp.log(s) - tgt_logit).reshape(1, 1)


def kernel(inputs, targets, features):
    n_tiles = pl.cdiv(_N, _TILE)
    tgt_rows = jnp.take(features, targets.astype(jnp.int32), axis=0)

    outputs, loss = pl.pallas_call(
        functools.partial(_fused_kernel, n_tiles=n_tiles),
        grid=(n_tiles,),
        in_specs=[
            pl.BlockSpec((_BATCH, _FEAT), lambda j: (0, 0)),
            pl.BlockSpec((_TILE, _FEAT), lambda j: (j, 0)),
            pl.BlockSpec((_BATCH, _FEAT), lambda j: (0, 0)),
        ],
        out_specs=[
            pl.BlockSpec((_BATCH, _TILE), lambda j: (0, j)),
            pl.BlockSpec((1, 1), lambda j: (0, 0)),
        ],
        out_shape=[
            jax.ShapeDtypeStruct((_BATCH, _N), jnp.float32),
            jax.ShapeDtypeStruct((1, 1), jnp.float32),
        ],
        scratch_shapes=[
            pltpu.VMEM((_BATCH, _FEAT), jnp.float32),
            pltpu.VMEM((_BATCH, 1), jnp.float32),
        ],
    )(inputs, features, tgt_rows)

    return (loss.reshape(()), outputs)
